# baseline (speedup 1.0000x reference)
import functools

import jax
import jax.numpy as jnp
from jax import lax
from jax.experimental import pallas as pl
from jax.experimental.pallas import tpu as pltpu
from jax.experimental.pallas import tpu_sc as plsc

N_USERS = 1000000
N_NODES = 16384
N_NEIGH = 50
DIM = 64
DIM2 = 2 * DIM
N_SLABS = N_NEIGH + 1
TOTAL_ROWS = N_SLABS * N_NODES
HALF_V = N_USERS // 2

NC, NS = 2, 16
NW = NC * NS
ROWS_PER_W = TOTAL_ROWS // NW
CHUNK = 96

REPACK_BP = 4096
REPACK_GRID = 62
H_STRIP = REPACK_BP * REPACK_GRID
STRIP_OFF = (0, 62, 124, 183)
STRIP_BASE = tuple(o * REPACK_BP for o in STRIP_OFF)


@functools.cache
def _make_sc_gather(total_rows):
    mesh = plsc.VectorSubcoreMesh(core_axis_name="c", subcore_axis_name="s")
    rows_per_w = total_rows // NW
    n_chunks = rows_per_w // CHUNK
    assert n_chunks % 2 == 0 and n_chunks >= 4

    @functools.partial(
        pl.kernel,
        out_type=jax.ShapeDtypeStruct((total_rows, DIM), jnp.float32),
        mesh=mesh,
        scratch_types=[
            pltpu.VMEM((rows_per_w,), jnp.int32),
            pltpu.VMEM((CHUNK, DIM), jnp.float32),
            pltpu.VMEM((CHUNK, DIM), jnp.float32),
            pltpu.SemaphoreType.DMA,
            pltpu.SemaphoreType.DMA,
            pltpu.SemaphoreType.DMA,
            pltpu.SemaphoreType.DMA,
        ],
        compiler_params=pltpu.CompilerParams(use_tc_tiling_on_sc=False),
    )
    def sc_gather(idx_hbm, table_hbm, out_hbm, idx_v, rows0, rows1,
                  semg0, semg1, semw0, semw1):
        wid = lax.axis_index("s") * NC + lax.axis_index("c")
        base = wid * rows_per_w
        rows = (rows0, rows1)
        semg = (semg0, semg1)
        semw = (semw0, semw1)

        pltpu.sync_copy(idx_hbm.at[pl.ds(base, rows_per_w)], idx_v)

        def start_g(i, j):
            pltpu.async_copy(
                table_hbm.at[idx_v.at[pl.ds(i * CHUNK, CHUNK)]],
                rows[j], semg[j])

        def wait_g(j):
            pltpu.make_async_copy(
                table_hbm.at[idx_v.at[pl.ds(0, CHUNK)]], rows[j],
                semg[j]).wait()

        def start_w(i, j):
            pltpu.async_copy(
                rows[j], out_hbm.at[pl.ds(base + i * CHUNK, CHUNK)], semw[j])

        def wait_w(j):
            pltpu.make_async_copy(
                table_hbm.at[idx_v.at[pl.ds(0, CHUNK)]], rows[j],
                semw[j]).wait()

        start_g(0, 0)
        start_g(1, 1)

        def body(io, carry):
            for j in (0, 1):
                i = 2 * io + j
                wait_g(j)
                start_w(i, j)
                wait_w(j)
                start_g(i + 2, j)
            return carry

        lax.fori_loop(0, n_chunks // 2 - 1, body, 0)
        for j in (0, 1):
            i = n_chunks - 2 + j
            wait_g(j)
            start_w(i, j)
            wait_w(j)

    return sc_gather


def _repack_body(a_ref, b_ref, c_ref, d_ref, eye_ref, o_ref):
    eye = eye_ref[...]
    tp = lambda r: jnp.dot(r[...].T, eye, preferred_element_type=jnp.float32)
    ab = jnp.concatenate([tp(a_ref), tp(b_ref)], axis=1)
    cd = jnp.concatenate([tp(c_ref), tp(d_ref)], axis=1)
    o_ref[...] = jnp.concatenate([ab, cd], axis=0)


def _repack(table_t):
    return pl.pallas_call(
        _repack_body,
        grid=(REPACK_GRID,),
        in_specs=[
            pl.BlockSpec((DIM, REPACK_BP), lambda i, o=o: (0, i + o))
            for o in STRIP_OFF
        ] + [pl.BlockSpec((DIM, DIM), lambda i: (0, 0))],
        out_specs=pl.BlockSpec((2 * REPACK_BP, DIM2), lambda i: (i, 0)),
        out_shape=jax.ShapeDtypeStruct((2 * H_STRIP, DIM2), jnp.float32),
        compiler_params=pltpu.CompilerParams(vmem_limit_bytes=100 * 1024 * 1024),
    )(table_t, table_t, table_t, table_t, jnp.eye(DIM, dtype=jnp.float32))


def _mlp_body(e_ref, w1e_ref, w1u_ref, b1_ref, w2_ref, b2_ref, w3_ref,
              ep_ref, l1u_ref, l1n_ref, bl_ref, o_ref):
    k = e_ref.shape[0] - 1
    bp = e_ref.shape[1]
    arr = e_ref[...]
    u = arr[0]
    x3 = arr[1:]
    x = x3.reshape(k * bp, DIM2)

    t = jnp.dot(u, w1u_ref[...], preferred_element_type=jnp.float32)
    tb = jnp.broadcast_to(t[None], (k, bp, DIM2)).reshape(k * bp, DIM2)
    h1 = jnp.maximum(
        jnp.dot(x, w1e_ref[...], preferred_element_type=jnp.float32)
        + tb + b1_ref[...], 0.0)
    h2 = jnp.maximum(
        jnp.dot(h1, w2_ref[...], preferred_element_type=jnp.float32)
        + b2_ref[...], 0.0)
    s = jnp.dot(h2, w3_ref[...], preferred_element_type=jnp.float32)

    s3 = s.reshape(k, bp, 2)
    m = jnp.max(s3, axis=0, keepdims=True)
    ex = jnp.exp(s3 - m)
    den = jnp.sum(ex, axis=0, keepdims=True)
    att = (ex / den).reshape(k * bp, 2)
    attb = jnp.dot(att, ep_ref[...],
                   preferred_element_type=jnp.float32).reshape(k, bp, DIM2)
    neigh = jnp.sum(x3 * attb, axis=0)

    out = jnp.maximum(
        jnp.dot(u, l1u_ref[...], preferred_element_type=jnp.float32)
        + jnp.dot(neigh, l1n_ref[...], preferred_element_type=jnp.float32)
        + bl_ref[...], 0.0)
    o_ref[...] = out


def _tc_mlp(e3, w1e, w1u, b1, w2t, b2, w3, ep, l1u, l1n, bl, bp):
    npair = e3.shape[1]
    grid = npair // bp
    full = lambda shape: pl.BlockSpec(shape, lambda i: (0,) * len(shape))
    return pl.pallas_call(
        _mlp_body,
        grid=(grid,),
        in_specs=[
            pl.BlockSpec((N_SLABS, bp, DIM2), lambda i: (0, i, 0)),
            full((DIM2, DIM2)), full((DIM2, DIM2)), full((1, DIM2)),
            full((DIM2, DIM2)), full((1, DIM2)), full((DIM2, 2)),
            full((2, DIM2)),
            full((DIM2, DIM2)), full((DIM2, DIM2)), full((1, DIM2)),
        ],
        out_specs=pl.BlockSpec((bp, DIM2), lambda i: (i, 0)),
        out_shape=jax.ShapeDtypeStruct((npair, DIM2), jnp.float32),
    )(e3, w1e, w1u, b1, w2t, b2, w3, ep, l1u, l1n, bl)


def _blockdiag(w):
    z = jnp.zeros((DIM, DIM), jnp.float32)
    return jnp.concatenate([
        jnp.concatenate([w, z], axis=1),
        jnp.concatenate([z, w], axis=1),
    ], axis=0)


def _pair2(b):
    return jnp.concatenate([b, b]).reshape(1, DIM2)


def _remap_idx(u):
    q = jnp.where(u < STRIP_BASE[1], 0,
                  jnp.where(u < STRIP_BASE[2], 1,
                            jnp.where(u < STRIP_BASE[2] + H_STRIP, 2, 3)))
    base = jnp.where(q == 0, 0,
                     jnp.where(q == 1, STRIP_BASE[1],
                               jnp.where(q == 2, STRIP_BASE[2],
                                         STRIP_BASE[3])))
    d = u - base
    bp_bits = REPACK_BP.bit_length() - 1
    return (((d >> bp_bits) << (bp_bits + 2)) + ((q >> 1) << (bp_bits + 1))
            + ((d & (REPACK_BP - 1)) << 1) + (q & 1))


N_SPLIT = 4


def kernel(nodes, neighbors, u2e_weight, att1_W, att1_b, att2_W, att2_b,
           att3_W, att3_b, lin1_W, lin1_b):
    del att3_b
    repacked = _repack(u2e_weight.T)
    table = repacked.reshape(4 * H_STRIP, DIM)
    nodes = nodes.astype(jnp.int32)
    neighbors = neighbors.astype(jnp.int32)
    nh = N_NODES // N_SPLIT
    gathers = []
    for c in range(N_SPLIT):
        sl = slice(c * nh, (c + 1) * nh)
        idx_c = jnp.concatenate(
            [nodes[sl], neighbors[sl].T.reshape(-1)])
        g = _make_sc_gather(N_SLABS * nh)(_remap_idx(idx_c), table)
        gathers.append(g.reshape(N_SLABS, nh // 2, DIM2))

    w1e = _blockdiag(att1_W[:, :DIM].T)
    w1u = _blockdiag(att1_W[:, DIM:].T)
    w2t = _blockdiag(att2_W.T)
    w3c = att3_W.reshape(DIM, 1)
    zc = jnp.zeros((DIM, 1), jnp.float32)
    w3 = jnp.concatenate([
        jnp.concatenate([w3c, zc], axis=1),
        jnp.concatenate([zc, w3c], axis=1),
    ], axis=0)
    one = jnp.ones((1, DIM), jnp.float32)
    zr = jnp.zeros((1, DIM), jnp.float32)
    ep = jnp.concatenate([
        jnp.concatenate([one, zr], axis=1),
        jnp.concatenate([zr, one], axis=1),
    ], axis=0)
    l1u = _blockdiag(lin1_W[:, :DIM].T)
    l1n = _blockdiag(lin1_W[:, DIM:].T)

    outs = [
        _tc_mlp(e3, w1e, w1u, _pair2(att1_b), w2t, _pair2(att2_b),
                w3, ep, l1u, l1n, _pair2(lin1_b), bp=256).reshape(-1, DIM)
        for e3 in gathers
    ]
    return jnp.concatenate(outs, axis=0)


# speedup vs baseline: 1.0663x; 1.0663x over previous
import functools

import jax
import jax.numpy as jnp
from jax import lax
from jax.experimental import pallas as pl
from jax.experimental.pallas import tpu as pltpu
from jax.experimental.pallas import tpu_sc as plsc

N_USERS = 1000000
N_NODES = 16384
N_NEIGH = 50
DIM = 64
DIM2 = 2 * DIM
N_SLABS = N_NEIGH + 1
TOTAL_ROWS = N_SLABS * N_NODES
HALF_V = N_USERS // 2

NC, NS = 2, 16
NW = NC * NS
ROWS_PER_W = TOTAL_ROWS // NW
CHUNK = 96

REPACK_BP = 8192
REPACK_GRID = 31
H_STRIP = REPACK_BP * REPACK_GRID
STRIP_OFF = (0, 31, 62, 92)
STRIP_BASE = tuple(o * REPACK_BP for o in STRIP_OFF)


@functools.cache
def _make_sc_gather(total_rows):
    mesh = plsc.VectorSubcoreMesh(core_axis_name="c", subcore_axis_name="s")
    rows_per_w = total_rows // NW
    n_chunks = rows_per_w // CHUNK
    assert (n_chunks - 4) % 4 == 0 and n_chunks >= 8

    @functools.partial(
        pl.kernel,
        out_type=jax.ShapeDtypeStruct((total_rows, DIM), jnp.float32),
        mesh=mesh,
        scratch_types=[
            pltpu.VMEM((rows_per_w,), jnp.int32),
            pltpu.VMEM((CHUNK, DIM), jnp.float32),
            pltpu.VMEM((CHUNK, DIM), jnp.float32),
            pltpu.VMEM((CHUNK, DIM), jnp.float32),
            pltpu.VMEM((CHUNK, DIM), jnp.float32),
            pltpu.SemaphoreType.DMA,
            pltpu.SemaphoreType.DMA,
            pltpu.SemaphoreType.DMA,
            pltpu.SemaphoreType.DMA,
            pltpu.SemaphoreType.DMA,
            pltpu.SemaphoreType.DMA,
            pltpu.SemaphoreType.DMA,
            pltpu.SemaphoreType.DMA,
        ],
        compiler_params=pltpu.CompilerParams(use_tc_tiling_on_sc=False),
    )
    def sc_gather(idx_hbm, table_hbm, out_hbm, idx_v, rows0, rows1, rows2,
                  rows3, semg0, semg1, semg2, semg3, semw0, semw1, semw2,
                  semw3):
        wid = lax.axis_index("s") * NC + lax.axis_index("c")
        base = wid * rows_per_w
        rows = (rows0, rows1, rows2, rows3)
        semg = (semg0, semg1, semg2, semg3)
        semw = (semw0, semw1, semw2, semw3)

        pltpu.sync_copy(idx_hbm.at[pl.ds(base, rows_per_w)], idx_v)

        def start_g(i, j):
            pltpu.async_copy(
                table_hbm.at[idx_v.at[pl.ds(i * CHUNK, CHUNK)]],
                rows[j], semg[j])

        def wait_g(j):
            pltpu.make_async_copy(
                table_hbm.at[idx_v.at[pl.ds(0, CHUNK)]], rows[j],
                semg[j]).wait()

        def start_w(i, j):
            pltpu.async_copy(
                rows[j], out_hbm.at[pl.ds(base + i * CHUNK, CHUNK)], semw[j])

        def wait_w(j):
            pltpu.make_async_copy(
                table_hbm.at[idx_v.at[pl.ds(0, CHUNK)]], rows[j],
                semw[j]).wait()

        start_g(0, 0)
        start_g(1, 1)
        wait_g(0)
        start_w(0, 0)
        start_g(2, 2)
        wait_g(1)
        start_w(1, 1)
        start_g(3, 3)

        def body(io, carry):
            for jj in range(4):
                i = 2 + 4 * io + jj
                j = (2 + jj) % 4
                wait_g(j)
                start_w(i, j)
                wait_w(jj)
                start_g(i + 2, jj)
            return carry

        lax.fori_loop(0, (n_chunks - 4) // 4, body, 0)
        for i in (n_chunks - 2, n_chunks - 1):
            j = i % 4
            wait_g(j)
            start_w(i, j)
        for j in range(4):
            wait_w(j)

    return sc_gather


def _repack_body(a_ref, b_ref, c_ref, d_ref, eye_ref, o_ref):
    eye = eye_ref[...]
    tp = lambda r: jnp.dot(r[...].T, eye, preferred_element_type=jnp.float32)
    ab = jnp.concatenate([tp(a_ref), tp(b_ref)], axis=1)
    cd = jnp.concatenate([tp(c_ref), tp(d_ref)], axis=1)
    o_ref[...] = jnp.concatenate([ab, cd], axis=0)


def _repack(table_t):
    return pl.pallas_call(
        _repack_body,
        grid=(REPACK_GRID,),
        in_specs=[
            pl.BlockSpec((DIM, REPACK_BP), lambda i, o=o: (0, i + o))
            for o in STRIP_OFF
        ] + [pl.BlockSpec((DIM, DIM), lambda i: (0, 0))],
        out_specs=pl.BlockSpec((2 * REPACK_BP, DIM2), lambda i: (i, 0)),
        out_shape=jax.ShapeDtypeStruct((2 * H_STRIP, DIM2), jnp.float32),
        compiler_params=pltpu.CompilerParams(vmem_limit_bytes=100 * 1024 * 1024),
    )(table_t, table_t, table_t, table_t, jnp.eye(DIM, dtype=jnp.float32))


def _mlp_body(e_ref, w1e_ref, w1u_ref, b1_ref, w2_ref, b2_ref, w3_ref,
              ep_ref, l1u_ref, l1n_ref, bl_ref, o_ref):
    k = e_ref.shape[0] - 1
    bp = e_ref.shape[1]
    arr = e_ref[...]
    u = arr[0]
    x3 = arr[1:]
    x = x3.reshape(k * bp, DIM2)

    t = jnp.dot(u, w1u_ref[...], preferred_element_type=jnp.float32)
    tb = jnp.broadcast_to(t[None], (k, bp, DIM2)).reshape(k * bp, DIM2)
    h1 = jnp.maximum(
        jnp.dot(x, w1e_ref[...], preferred_element_type=jnp.float32)
        + tb + b1_ref[...], 0.0)
    h2 = jnp.maximum(
        jnp.dot(h1, w2_ref[...], preferred_element_type=jnp.float32)
        + b2_ref[...], 0.0)
    s = jnp.dot(h2, w3_ref[...], preferred_element_type=jnp.float32)

    s3 = s.reshape(k, bp, 2)
    m = jnp.max(s3, axis=0, keepdims=True)
    ex = jnp.exp(s3 - m)
    den = jnp.sum(ex, axis=0, keepdims=True)
    att = (ex / den).reshape(k * bp, 2)
    attb = jnp.dot(att, ep_ref[...],
                   preferred_element_type=jnp.float32).reshape(k, bp, DIM2)
    neigh = jnp.sum(x3 * attb, axis=0)

    out = jnp.maximum(
        jnp.dot(u, l1u_ref[...], preferred_element_type=jnp.float32)
        + jnp.dot(neigh, l1n_ref[...], preferred_element_type=jnp.float32)
        + bl_ref[...], 0.0)
    o_ref[...] = out


def _tc_mlp(e3, w1e, w1u, b1, w2t, b2, w3, ep, l1u, l1n, bl, bp):
    npair = e3.shape[1]
    grid = npair // bp
    full = lambda shape: pl.BlockSpec(shape, lambda i: (0,) * len(shape))
    return pl.pallas_call(
        _mlp_body,
        grid=(grid,),
        in_specs=[
            pl.BlockSpec((N_SLABS, bp, DIM2), lambda i: (0, i, 0)),
            full((DIM2, DIM2)), full((DIM2, DIM2)), full((1, DIM2)),
            full((DIM2, DIM2)), full((1, DIM2)), full((DIM2, 2)),
            full((2, DIM2)),
            full((DIM2, DIM2)), full((DIM2, DIM2)), full((1, DIM2)),
        ],
        out_specs=pl.BlockSpec((bp, DIM2), lambda i: (i, 0)),
        out_shape=jax.ShapeDtypeStruct((npair, DIM2), jnp.float32),
    )(e3, w1e, w1u, b1, w2t, b2, w3, ep, l1u, l1n, bl)


def _blockdiag(w):
    z = jnp.zeros((DIM, DIM), jnp.float32)
    return jnp.concatenate([
        jnp.concatenate([w, z], axis=1),
        jnp.concatenate([z, w], axis=1),
    ], axis=0)


def _pair2(b):
    return jnp.concatenate([b, b]).reshape(1, DIM2)


def _remap_idx(u):
    q = jnp.where(u < STRIP_BASE[1], 0,
                  jnp.where(u < STRIP_BASE[2], 1,
                            jnp.where(u < STRIP_BASE[2] + H_STRIP, 2, 3)))
    base = jnp.where(q == 0, 0,
                     jnp.where(q == 1, STRIP_BASE[1],
                               jnp.where(q == 2, STRIP_BASE[2],
                                         STRIP_BASE[3])))
    d = u - base
    bp_bits = REPACK_BP.bit_length() - 1
    return (((d >> bp_bits) << (bp_bits + 2)) + ((q >> 1) << (bp_bits + 1))
            + ((d & (REPACK_BP - 1)) << 1) + (q & 1))


N_SPLIT = 4


def kernel(nodes, neighbors, u2e_weight, att1_W, att1_b, att2_W, att2_b,
           att3_W, att3_b, lin1_W, lin1_b):
    del att3_b
    repacked = _repack(u2e_weight.T)
    table = repacked.reshape(4 * H_STRIP, DIM)
    nodes = nodes.astype(jnp.int32)
    neighbors = neighbors.astype(jnp.int32)
    nh = N_NODES // N_SPLIT
    gathers = []
    for c in range(N_SPLIT):
        sl = slice(c * nh, (c + 1) * nh)
        idx_c = jnp.concatenate(
            [nodes[sl], neighbors[sl].T.reshape(-1)])
        g = _make_sc_gather(N_SLABS * nh)(_remap_idx(idx_c), table)
        gathers.append(g.reshape(N_SLABS, nh // 2, DIM2))

    w1e = _blockdiag(att1_W[:, :DIM].T)
    w1u = _blockdiag(att1_W[:, DIM:].T)
    w2t = _blockdiag(att2_W.T)
    w3c = att3_W.reshape(DIM, 1)
    zc = jnp.zeros((DIM, 1), jnp.float32)
    w3 = jnp.concatenate([
        jnp.concatenate([w3c, zc], axis=1),
        jnp.concatenate([zc, w3c], axis=1),
    ], axis=0)
    one = jnp.ones((1, DIM), jnp.float32)
    zr = jnp.zeros((1, DIM), jnp.float32)
    ep = jnp.concatenate([
        jnp.concatenate([one, zr], axis=1),
        jnp.concatenate([zr, one], axis=1),
    ], axis=0)
    l1u = _blockdiag(lin1_W[:, :DIM].T)
    l1n = _blockdiag(lin1_W[:, DIM:].T)

    outs = [
        _tc_mlp(e3, w1e, w1u, _pair2(att1_b), w2t, _pair2(att2_b),
                w3, ep, l1u, l1n, _pair2(lin1_b), bp=256).reshape(-1, DIM)
        for e3 in gathers
    ]
    return jnp.concatenate(outs, axis=0)
